# baseline (speedup 1.0000x reference)
import functools

import jax
import jax.numpy as jnp
from jax import lax
from jax.experimental import pallas as pl
from jax.experimental.pallas import tpu as pltpu
from jax.experimental.pallas import tpu_sc as plsc

_F = 26
_V = 100000
_D = 32
_B = 16384

_NC = 2
_NS = 16
_NW = _NC * _NS

_POS = _B * _F
_PW = _POS // _NW
_SEG = 128
_C = 1664
_NSEG = _C // _SEG
_NCHUNK = _PW // _C


def _make_kernel():
    mesh = plsc.VectorSubcoreMesh(core_axis_name="c", subcore_axis_name="s")

    @functools.partial(
        pl.kernel,
        mesh=mesh,
        out_type=jax.ShapeDtypeStruct((_POS, _D), jnp.float32),
        compiler_params=pltpu.CompilerParams(use_tc_tiling_on_sc=False),
        scratch_types=[
            pltpu.VMEM((_C,), jnp.int32),
            pltpu.VMEM((_NSEG, _SEG), jnp.int32),
            pltpu.VMEM((_C,), jnp.int32),
            pltpu.VMEM((_C, _D), jnp.float32),
            pltpu.SemaphoreType.DMA,
        ],
    )
    def emb(x_hbm, offs_hbm, table_hbm, out_hbm, xraw_v, idx_v, offs_v,
            rows_v, sem):
        wid = lax.axis_index("s") * _NC + lax.axis_index("c")
        wbase = wid * _PW

        pltpu.sync_copy(offs_hbm, offs_v)

        def chunk_body(i, carry):
            row_base = wbase + i * _C
            pltpu.sync_copy(x_hbm.at[pl.ds(row_base, _C)], xraw_v)
            for k in range(_NSEG):
                for j in range(_SEG // 16):
                    sl = pl.ds(k * _SEG + j * 16, 16)
                    idx_v[k, pl.ds(j * 16, 16)] = xraw_v[sl] + offs_v[sl]
            copies = []
            for k in range(_NSEG):
                copies.append(
                    pltpu.async_copy(
                        table_hbm.at[idx_v.at[k]],
                        rows_v.at[pl.ds(k * _SEG, _SEG)],
                        sem,
                    ))
            for c in copies:
                c.wait()
            pltpu.sync_copy(rows_v, out_hbm.at[pl.ds(row_base, _C)])
            return carry

        lax.fori_loop(0, _NCHUNK, chunk_body, 0)

    return emb


_emb_kernel = _make_kernel()


def kernel(X, tables):
    x_flat = X.reshape(_POS).astype(jnp.int32)
    offs = jnp.tile(jnp.arange(_F, dtype=jnp.int32) * _V, _C // _F)
    table_flat = tables.reshape(_F * _V, _D)
    out = _emb_kernel(x_flat, offs, table_flat)
    return out.reshape(_B, _F, _D)


# speedup vs baseline: 1.7798x; 1.7798x over previous
import functools

import jax
import jax.numpy as jnp
from jax import lax
from jax.experimental import pallas as pl
from jax.experimental.pallas import tpu as pltpu
from jax.experimental.pallas import tpu_sc as plsc

_F = 26
_V = 100000
_D = 32
_B = 16384

_NC = 2
_NS = 16
_NW = _NC * _NS

_R = _F * _D
_RPW = _R // _NW
_VEC = 16
_UNROLL = 8
_NITER = _B // (_VEC * _UNROLL)


def _make_kernel():
    mesh = plsc.VectorSubcoreMesh(core_axis_name="c", subcore_axis_name="s")

    @functools.partial(
        pl.kernel,
        mesh=mesh,
        out_type=jax.ShapeDtypeStruct((_R, _B), jnp.float32),
        compiler_params=pltpu.CompilerParams(
            use_tc_tiling_on_sc=False, needs_layout_passes=False),
        scratch_types=[
            pltpu.VMEM((_V,), jnp.float32),
            pltpu.VMEM((_B,), jnp.float32),
            pltpu.SemaphoreType.DMA,
            pltpu.SemaphoreType.DMA,
        ],
    )
    def emb(tv_hbm, x_hbm, out_hbm, row_v, buf_v, sem_r, sem_x):
        wid = lax.axis_index("s") * _NC + lax.axis_index("c")
        r0 = wid * _RPW

        def row_body(rl, carry):
            r = r0 + rl
            f = r // _D
            cp_r = pltpu.async_copy(tv_hbm.at[r], row_v, sem_r)
            cp_x = pltpu.async_copy(x_hbm.at[pl.ds(f * _B, _B)], buf_v, sem_x)
            cp_r.wait()
            cp_x.wait()

            def gather_body(i, c2):
                base = i * (_VEC * _UNROLL)
                for u in range(_UNROLL):
                    sl = pl.ds(base + u * _VEC, _VEC)
                    xi = plsc.bitcast(buf_v[sl], jnp.int32)
                    buf_v[sl] = plsc.load_gather(row_v, [xi])
                return c2

            lax.fori_loop(0, _NITER, gather_body, 0)
            pltpu.sync_copy(buf_v, out_hbm.at[r])
            return carry

        lax.fori_loop(0, _RPW, row_body, 0)

    return emb


_emb_kernel = _make_kernel()


def kernel(X, tables):
    tv = jnp.transpose(tables, (0, 2, 1)).reshape(_R, _V)
    xt = jnp.transpose(X.astype(jnp.int32), (1, 0)).reshape(_F * _B)
    xf = lax.bitcast_convert_type(xt, jnp.float32)
    out = _emb_kernel(tv, xf)
    return jnp.transpose(out.reshape(_F, _D, _B), (2, 0, 1))
